# baseline (speedup 1.0000x reference)
import functools

import jax
import jax.numpy as jnp
from jax import lax
from jax.experimental import pallas as pl
from jax.experimental.pallas import tpu as pltpu
from jax.experimental.pallas import tpu_sc as plsc

_N = 10000
_E = 320000
_B = 64
_DN = 128
_DE = 16
_H = 64
_OUT = 32
_T = 12

_NC = 2
_NS = 16
_NW = _NC * _NS
_C = 128
_NCHUNKS = _E // _C


def _edge_mlp_body(ef_ref, w1, b1, w2, b2, o_ref):
    ef = ef_ref[...]
    h1 = jnp.maximum(jnp.dot(ef, w1[...], preferred_element_type=jnp.float32) + b1[...], 0.0)
    o_ref[...] = jnp.dot(h1, w2[...], preferred_element_type=jnp.float32) + b2[...]


def _edge_mlp(ef, w1, b1, w2, b2):
    be = 4000
    h = w2.shape[1]
    full = lambda shp: pl.BlockSpec(shp, lambda i: (0, 0))
    return pl.pallas_call(
        _edge_mlp_body,
        grid=(_E // be,),
        in_specs=[
            pl.BlockSpec((be, _DE), lambda i: (i, 0)),
            full((_DE, h)), full((1, h)), full((h, h)), full((1, h)),
        ],
        out_specs=pl.BlockSpec((be, h), lambda i: (i, 0)),
        out_shape=jax.ShapeDtypeStruct((_E, h), jnp.float32),
    )(ef, w1, b1.reshape(1, -1), w2, b2.reshape(1, -1))


def _mm_body(x_ref, w_ref, o_ref):
    o_ref[...] = jnp.dot(x_ref[...], w_ref[...], preferred_element_type=jnp.float32)


def _matmul(x, w):
    n, d = x.shape
    h = w.shape[1]
    return pl.pallas_call(
        _mm_body,
        out_shape=jax.ShapeDtypeStruct((n, h), jnp.float32),
    )(x, w)


def _relu_mm_body(ya_ref, yb_ref, b_ref, w_ref, o_ref):
    x = jnp.maximum(ya_ref[...] + yb_ref[...] + b_ref[...], 0.0)
    o_ref[...] = jnp.dot(x, w_ref[...], preferred_element_type=jnp.float32)


def _relu_mm(ya, yb, b, w):
    n, h = ya.shape
    h2 = w.shape[1]
    return pl.pallas_call(
        _relu_mm_body,
        out_shape=jax.ShapeDtypeStruct((n, h2), jnp.float32),
    )(ya, yb, b.reshape(1, -1), w)


def _set2set_body(ya_ref, yb_ref, bo_ref, batch_ref, wi_ref, wh_ref, bb_ref, o_ref):
    x = ya_ref[...] + yb_ref[...] + bo_ref[...]
    wi = wi_ref[...]
    wh = wh_ref[...]
    bb = bb_ref[...]
    batch = batch_ref[...]
    seg = lax.broadcasted_iota(jnp.int32, (_B, _N), 0)
    mask = jnp.broadcast_to(batch, (_B, _N)) == seg

    q_star = jnp.zeros((_B, 2 * _OUT), jnp.float32)
    h = jnp.zeros((_B, _OUT), jnp.float32)
    c = jnp.zeros((_B, _OUT), jnp.float32)
    for t in range(_T):
        gates = (jnp.dot(q_star, wi, preferred_element_type=jnp.float32)
                 + jnp.dot(h, wh, preferred_element_type=jnp.float32) + bb)
        ig = jax.nn.sigmoid(gates[:, 0 * _OUT:1 * _OUT])
        fg = jax.nn.sigmoid(gates[:, 1 * _OUT:2 * _OUT])
        gg = jnp.tanh(gates[:, 2 * _OUT:3 * _OUT])
        og = jax.nn.sigmoid(gates[:, 3 * _OUT:4 * _OUT])
        c = fg * c + ig * gg
        h = og * jnp.tanh(c)
        if t == _T - 1:
            break
        s = lax.dot_general(h, x, (((1,), (1,)), ((), ())),
                            preferred_element_type=jnp.float32)
        sm = jnp.where(mask, s, -1e30)
        m = jnp.max(sm, axis=1, keepdims=True)
        ex = jnp.exp(jnp.where(mask, s - m, -1e30))
        den = jnp.sum(ex, axis=1, keepdims=True)
        a = ex / (den + 1e-16)
        r = jnp.dot(a, x, preferred_element_type=jnp.float32)
        q_star = jnp.concatenate([h, r], axis=1)
    o_ref[...] = h


def _set2set(ya, yb, b_out, batch, wi, wh, bsum):
    return pl.pallas_call(
        _set2set_body,
        out_shape=jax.ShapeDtypeStruct((_B, _OUT), jnp.float32),
    )(ya, yb, b_out.reshape(1, -1), batch.reshape(1, -1), wi, wh,
      bsum.reshape(1, -1))


@functools.lru_cache(maxsize=None)
def _make_egc_sc(h):
    rows_per_sub = 632
    base_chunks = _NCHUNKS // _NW
    rem = _NCHUNKS % _NW
    nlmax = base_chunks + 1
    mesh = plsc.VectorSubcoreMesh(core_axis_name="c", subcore_axis_name="s",
                                  num_cores=_NC, num_subcores=_NS)

    @functools.partial(
        pl.kernel,
        out_type=jax.ShapeDtypeStruct((_NC, _N, h), jnp.float32),
        mesh=mesh,
        scratch_types=[
            pltpu.VMEM((nlmax, _C), jnp.int32),
            pltpu.VMEM((nlmax, _C), jnp.int32),
            pltpu.VMEM((_C, h), jnp.float32),
            pltpu.VMEM((_C, h), jnp.float32),
            pltpu.VMEM((_C, h), jnp.float32),
            pltpu.VMEM((_C, h), jnp.float32),
            pltpu.VMEM_SHARED((_N, h), jnp.float32),
            pltpu.SemaphoreType.DMA,
            pltpu.SemaphoreType.DMA,
            pltpu.SemaphoreType.DMA,
            pltpu.SemaphoreType.DMA,
        ],
        compiler_params=pltpu.CompilerParams(use_tc_tiling_on_sc=False),
    )
    def egc(xw_hbm, ef_hbm, esrc_hbm, etgt_hbm, zeros_hbm, out_hbm,
            esrc_l, etgt_l, ef0, ef1, rows0, rows1, acc_sh,
            sem_e0, sem_e1, sem_g0, sem_g1):
        ef_b = (ef0, ef1)
        rows_b = (rows0, rows1)
        sem_e = (sem_e0, sem_e1)
        sem_g = (sem_g0, sem_g1)
        cid = lax.axis_index("c")
        sid = lax.axis_index("s")
        wid = sid * _NC + cid
        row0 = pl.multiple_of(jnp.minimum(sid * rows_per_sub, _N - rows_per_sub), 8)
        pltpu.sync_copy(zeros_hbm.at[pl.ds(row0, rows_per_sub)],
                        acc_sh.at[pl.ds(row0, rows_per_sub)])

        nloc = base_chunks + jnp.where(wid < rem, 1, 0)
        c0 = wid * base_chunks + jnp.minimum(wid, rem)
        c0r = jnp.minimum(c0, _NCHUNKS - nlmax)
        off = c0 - c0r
        pltpu.sync_copy(esrc_hbm.at[pl.ds(c0r, nlmax)], esrc_l)
        pltpu.sync_copy(etgt_hbm.at[pl.ds(c0r, nlmax)], etgt_l)
        plsc.subcore_barrier()

        def fire(j, b):
            @pl.when(j < nloc)
            def _():
                base = (c0 + j) * _C
                pltpu.async_copy(ef_hbm.at[pl.ds(base, _C)], ef_b[b], sem_e[b])
                pltpu.async_copy(xw_hbm.at[esrc_l.at[j + off]], rows_b[b],
                                 sem_g[b])

        def process(j, b):
            @pl.when(j < nloc)
            def _():
                pltpu.make_async_copy(ef_hbm.at[pl.ds(0, _C)], ef_b[b],
                                      sem_e[b]).wait()
                pltpu.make_async_copy(xw_hbm.at[esrc_l.at[j + off]],
                                      rows_b[b], sem_g[b]).wait()

                @plsc.parallel_loop(0, _C, 1, unroll=4)
                def _mul(i):
                    for h0 in range(h // 16):
                        sl = pl.ds(h0 * 16, 16)
                        rows_b[b][i, sl] = rows_b[b][i, sl] * ef_b[b][i, sl]

                pltpu.sync_copy(rows_b[b], acc_sh.at[etgt_l.at[j + off]],
                                add=True)

        fire(0, 0)

        def outer(i, carry):
            j = 2 * i
            fire(j + 1, 1)
            process(j, 0)
            fire(j + 2, 0)
            process(j + 1, 1)
            return carry

        lax.fori_loop(0, (nlmax + 1) // 2, outer, 0)
        plsc.subcore_barrier()
        pltpu.sync_copy(acc_sh.at[pl.ds(row0, rows_per_sub)],
                        out_hbm.at[cid, pl.ds(row0, rows_per_sub)])

    return egc


def kernel(node_features, edge_features, Esrc, Etgt, batch,
           W_in, b_in, W_mid, b_mid, W_out, b_out,
           ee1_w1, ee1_b1, ee1_w2, ee1_b2,
           ee2_w1, ee2_b1, ee2_w2, ee2_b2,
           ee3_w1, ee3_b1, ee3_w2, ee3_b2,
           Wi, Wh, bi, bh):
    efin = _edge_mlp(edge_features, ee1_w1, ee1_b1, ee1_w2, ee1_b2)
    efmid = _edge_mlp(edge_features, ee2_w1, ee2_b1, ee2_w2, ee2_b2)
    efout = _edge_mlp(edge_features, ee3_w1, ee3_b1, ee3_w2, ee3_b2)

    zeros_h = jnp.zeros((_N, _H), jnp.float32)
    zeros_o = jnp.zeros((_N, _OUT), jnp.float32)
    esrc2 = Esrc.reshape(_NCHUNKS, _C)
    etgt2 = Etgt.reshape(_NCHUNKS, _C)

    xw = _matmul(node_features, W_in)
    y1 = _make_egc_sc(_H)(xw, efin, esrc2, etgt2, zeros_h)
    xw = _relu_mm(y1[0], y1[1], b_in, W_mid)
    y2 = _make_egc_sc(_H)(xw, efmid, esrc2, etgt2, zeros_h)
    xw = _relu_mm(y2[0], y2[1], b_mid, W_out)
    y3 = _make_egc_sc(_OUT)(xw, efout, esrc2, etgt2, zeros_o)
    return _set2set(y3[0], y3[1], b_out, batch, Wi, Wh, bi + bh)


# speedup vs baseline: 1.2328x; 1.2328x over previous
import functools

import jax
import jax.numpy as jnp
from jax import lax
from jax.experimental import pallas as pl
from jax.experimental.pallas import tpu as pltpu
from jax.experimental.pallas import tpu_sc as plsc

_N = 10000
_E = 320000
_B = 64
_DN = 128
_DE = 16
_H = 64
_OUT = 32
_T = 12

_NC = 2
_NS = 16
_NW = _NC * _NS
_C = 128
_NCHUNKS = _E // _C


def _edge_mlps_body(ef_ref, w11, b11, w12, b12, w21, b21, w22, b22,
                    w31, b31, w32, b32, x_ref, win_ref, o1, o2, o3, oxw):
    @pl.when(pl.program_id(0) == 0)
    def _():
        oxw[...] = jnp.dot(x_ref[...], win_ref[...],
                           preferred_element_type=jnp.float32)

    ef = ef_ref[...]
    h1 = jnp.maximum(jnp.dot(ef, w11[...], preferred_element_type=jnp.float32) + b11[...], 0.0)
    o1[...] = jnp.dot(h1, w12[...], preferred_element_type=jnp.float32) + b12[...]
    h2 = jnp.maximum(jnp.dot(ef, w21[...], preferred_element_type=jnp.float32) + b21[...], 0.0)
    o2[...] = jnp.dot(h2, w22[...], preferred_element_type=jnp.float32) + b22[...]
    h3 = jnp.maximum(jnp.dot(ef, w31[...], preferred_element_type=jnp.float32) + b31[...], 0.0)
    o3[...] = jnp.dot(h3, w32[...], preferred_element_type=jnp.float32) + b32[...]


def _edge_mlps(ef, w11, b11, w12, b12, w21, b21, w22, b22, w31, b31, w32, b32,
               x, w_in):
    be = 4000
    full = lambda shp: pl.BlockSpec(shp, lambda i: (0, 0))
    return pl.pallas_call(
        _edge_mlps_body,
        grid=(_E // be,),
        in_specs=[
            pl.BlockSpec((be, _DE), lambda i: (i, 0)),
            full((_DE, _H)), full((1, _H)), full((_H, _H)), full((1, _H)),
            full((_DE, _H)), full((1, _H)), full((_H, _H)), full((1, _H)),
            full((_DE, _OUT)), full((1, _OUT)), full((_OUT, _OUT)), full((1, _OUT)),
            full((_N, _DN)), full((_DN, _H)),
        ],
        out_specs=[
            pl.BlockSpec((be, _H), lambda i: (i, 0)),
            pl.BlockSpec((be, _H), lambda i: (i, 0)),
            pl.BlockSpec((be, _OUT), lambda i: (i, 0)),
            full((_N, _H)),
        ],
        out_shape=[
            jax.ShapeDtypeStruct((_E, _H), jnp.float32),
            jax.ShapeDtypeStruct((_E, _H), jnp.float32),
            jax.ShapeDtypeStruct((_E, _OUT), jnp.float32),
            jax.ShapeDtypeStruct((_N, _H), jnp.float32),
        ],
    )(ef, w11, b11.reshape(1, -1), w12, b12.reshape(1, -1),
      w21, b21.reshape(1, -1), w22, b22.reshape(1, -1),
      w31, b31.reshape(1, -1), w32, b32.reshape(1, -1), x, w_in)


def _mm_body(x_ref, w_ref, o_ref):
    o_ref[...] = jnp.dot(x_ref[...], w_ref[...], preferred_element_type=jnp.float32)


def _matmul(x, w):
    n, d = x.shape
    h = w.shape[1]
    return pl.pallas_call(
        _mm_body,
        out_shape=jax.ShapeDtypeStruct((n, h), jnp.float32),
    )(x, w)


def _relu_mm_body(ya_ref, yb_ref, b_ref, w_ref, o_ref):
    x = jnp.maximum(ya_ref[...] + yb_ref[...] + b_ref[...], 0.0)
    o_ref[...] = jnp.dot(x, w_ref[...], preferred_element_type=jnp.float32)


def _relu_mm(ya, yb, b, w):
    n, h = ya.shape
    h2 = w.shape[1]
    return pl.pallas_call(
        _relu_mm_body,
        out_shape=jax.ShapeDtypeStruct((n, h2), jnp.float32),
    )(ya, yb, b.reshape(1, -1), w)


def _set2set_body(ya_ref, yb_ref, bo_ref, batch_ref, wi_ref, wh_ref, bb_ref, o_ref):
    x = ya_ref[...] + yb_ref[...] + bo_ref[...]
    wi = wi_ref[...]
    wh = wh_ref[...]
    bb = bb_ref[...]
    batch = batch_ref[...]
    seg = lax.broadcasted_iota(jnp.int32, (_B, _N), 0)
    mask = jnp.broadcast_to(batch, (_B, _N)) == seg

    q_star = jnp.zeros((_B, 2 * _OUT), jnp.float32)
    h = jnp.zeros((_B, _OUT), jnp.float32)
    c = jnp.zeros((_B, _OUT), jnp.float32)
    for t in range(_T):
        gates = (jnp.dot(q_star, wi, preferred_element_type=jnp.float32)
                 + jnp.dot(h, wh, preferred_element_type=jnp.float32) + bb)
        ig = jax.nn.sigmoid(gates[:, 0 * _OUT:1 * _OUT])
        fg = jax.nn.sigmoid(gates[:, 1 * _OUT:2 * _OUT])
        gg = jnp.tanh(gates[:, 2 * _OUT:3 * _OUT])
        og = jax.nn.sigmoid(gates[:, 3 * _OUT:4 * _OUT])
        c = fg * c + ig * gg
        h = og * jnp.tanh(c)
        if t == _T - 1:
            break
        s = lax.dot_general(h, x, (((1,), (1,)), ((), ())),
                            preferred_element_type=jnp.float32)
        sm = jnp.where(mask, s, -1e30)
        m = jnp.max(sm, axis=1, keepdims=True)
        ex = jnp.exp(jnp.where(mask, s - m, -1e30))
        den = jnp.sum(ex, axis=1, keepdims=True)
        a = ex / (den + 1e-16)
        r = jnp.dot(a, x, preferred_element_type=jnp.float32)
        q_star = jnp.concatenate([h, r], axis=1)
    o_ref[...] = h


def _set2set(ya, yb, b_out, batch, wi, wh, bsum):
    return pl.pallas_call(
        _set2set_body,
        out_shape=jax.ShapeDtypeStruct((_B, _OUT), jnp.float32),
    )(ya, yb, b_out.reshape(1, -1), batch.reshape(1, -1), wi, wh,
      bsum.reshape(1, -1))


@functools.lru_cache(maxsize=None)
def _make_egc_sc(h):
    rows_per_sub = 632
    base_chunks = _NCHUNKS // _NW
    rem = _NCHUNKS % _NW
    nlmax = base_chunks + 1
    mesh = plsc.VectorSubcoreMesh(core_axis_name="c", subcore_axis_name="s",
                                  num_cores=_NC, num_subcores=_NS)

    @functools.partial(
        pl.kernel,
        out_type=jax.ShapeDtypeStruct((_NC, _N, h), jnp.float32),
        mesh=mesh,
        scratch_types=[
            pltpu.VMEM((nlmax, _C), jnp.int32),
            pltpu.VMEM((nlmax, _C), jnp.int32),
            pltpu.VMEM((_C, h), jnp.float32),
            pltpu.VMEM((_C, h), jnp.float32),
            pltpu.VMEM((_C, h), jnp.float32),
            pltpu.VMEM((_C, h), jnp.float32),
            pltpu.VMEM_SHARED((_N, h), jnp.float32),
            pltpu.SemaphoreType.DMA,
            pltpu.SemaphoreType.DMA,
            pltpu.SemaphoreType.DMA,
            pltpu.SemaphoreType.DMA,
        ],
        compiler_params=pltpu.CompilerParams(use_tc_tiling_on_sc=False),
    )
    def egc(xw_hbm, ef_hbm, esrc_hbm, etgt_hbm, zeros_hbm, out_hbm,
            esrc_l, etgt_l, ef0, ef1, rows0, rows1, acc_sh,
            sem_e0, sem_e1, sem_g0, sem_g1):
        ef_b = (ef0, ef1)
        rows_b = (rows0, rows1)
        sem_e = (sem_e0, sem_e1)
        sem_g = (sem_g0, sem_g1)
        cid = lax.axis_index("c")
        sid = lax.axis_index("s")
        wid = sid * _NC + cid
        row0 = pl.multiple_of(jnp.minimum(sid * rows_per_sub, _N - rows_per_sub), 8)
        pltpu.sync_copy(zeros_hbm.at[pl.ds(row0, rows_per_sub)],
                        acc_sh.at[pl.ds(row0, rows_per_sub)])

        nloc = base_chunks + jnp.where(wid < rem, 1, 0)
        c0 = wid * base_chunks + jnp.minimum(wid, rem)
        c0r = jnp.minimum(c0, _NCHUNKS - nlmax)
        off = c0 - c0r
        pltpu.sync_copy(esrc_hbm.at[pl.ds(c0r, nlmax)], esrc_l)
        pltpu.sync_copy(etgt_hbm.at[pl.ds(c0r, nlmax)], etgt_l)
        plsc.subcore_barrier()

        def fire(j, b):
            @pl.when(j < nloc)
            def _():
                base = (c0 + j) * _C
                pltpu.async_copy(ef_hbm.at[pl.ds(base, _C)], ef_b[b], sem_e[b])
                pltpu.async_copy(xw_hbm.at[esrc_l.at[j + off]], rows_b[b],
                                 sem_g[b])

        def process(j, b):
            @pl.when(j < nloc)
            def _():
                pltpu.make_async_copy(ef_hbm.at[pl.ds(0, _C)], ef_b[b],
                                      sem_e[b]).wait()
                pltpu.make_async_copy(xw_hbm.at[esrc_l.at[j + off]],
                                      rows_b[b], sem_g[b]).wait()

                @plsc.parallel_loop(0, _C, 1, unroll=4)
                def _mul(i):
                    for h0 in range(h // 16):
                        sl = pl.ds(h0 * 16, 16)
                        rows_b[b][i, sl] = rows_b[b][i, sl] * ef_b[b][i, sl]

                pltpu.sync_copy(rows_b[b], acc_sh.at[etgt_l.at[j + off]],
                                add=True)

        fire(0, 0)

        def outer(i, carry):
            j = 2 * i
            fire(j + 1, 1)
            process(j, 0)
            fire(j + 2, 0)
            process(j + 1, 1)
            return carry

        lax.fori_loop(0, (nlmax + 1) // 2, outer, 0)
        plsc.subcore_barrier()
        pltpu.sync_copy(acc_sh.at[pl.ds(row0, rows_per_sub)],
                        out_hbm.at[cid, pl.ds(row0, rows_per_sub)])

    return egc


def kernel(node_features, edge_features, Esrc, Etgt, batch,
           W_in, b_in, W_mid, b_mid, W_out, b_out,
           ee1_w1, ee1_b1, ee1_w2, ee1_b2,
           ee2_w1, ee2_b1, ee2_w2, ee2_b2,
           ee3_w1, ee3_b1, ee3_w2, ee3_b2,
           Wi, Wh, bi, bh):
    efin, efmid, efout, xw0 = _edge_mlps(
        edge_features,
        ee1_w1, ee1_b1, ee1_w2, ee1_b2,
        ee2_w1, ee2_b1, ee2_w2, ee2_b2,
        ee3_w1, ee3_b1, ee3_w2, ee3_b2,
        node_features, W_in)

    zeros_h = jnp.zeros((_N, _H), jnp.float32)
    zeros_o = jnp.zeros((_N, _OUT), jnp.float32)
    esrc2 = Esrc.reshape(_NCHUNKS, _C)
    etgt2 = Etgt.reshape(_NCHUNKS, _C)

    y1 = _make_egc_sc(_H)(xw0, efin, esrc2, etgt2, zeros_h)
    xw = _relu_mm(y1[0], y1[1], b_in, W_mid)
    y2 = _make_egc_sc(_H)(xw, efmid, esrc2, etgt2, zeros_h)
    xw = _relu_mm(y2[0], y2[1], b_mid, W_out)
    y3 = _make_egc_sc(_OUT)(xw, efout, esrc2, etgt2, zeros_o)
    return _set2set(y3[0], y3[1], b_out, batch, Wi, Wh, bi + bh)


# speedup vs baseline: 1.3752x; 1.1155x over previous
import functools

import jax
import jax.numpy as jnp
from jax import lax
from jax.experimental import pallas as pl
from jax.experimental.pallas import tpu as pltpu
from jax.experimental.pallas import tpu_sc as plsc

_N = 10000
_E = 320000
_B = 64
_DN = 128
_DE = 16
_H = 64
_OUT = 32
_T = 12

_NC = 2
_NS = 16
_NW = _NC * _NS
_C = 128
_NCHUNKS = _E // _C


def _blockdiag(w, k):
    d, h = w.shape
    out = jnp.zeros((k * d, k * h), jnp.float32)
    for i in range(k):
        out = out.at[i * d:(i + 1) * d, i * h:(i + 1) * h].set(w)
    return out


def _edge_mlps_body(ef2_ref, ef4_ref, w11, b11, w12, b12, w21, b21, w22, b22,
                    w31, b31, w32, b32, x_ref, win_ref, o1, o2, o3, oxw):
    @pl.when(pl.program_id(0) == 0)
    def _():
        oxw[...] = jnp.dot(x_ref[...], win_ref[...],
                           preferred_element_type=jnp.float32)

    ef2 = ef2_ref[...]
    ef4 = ef4_ref[...]
    h1 = jnp.maximum(jnp.dot(ef2, w11[...], preferred_element_type=jnp.float32) + b11[...], 0.0)
    o1[...] = jnp.dot(h1, w12[...], preferred_element_type=jnp.float32) + b12[...]
    h2 = jnp.maximum(jnp.dot(ef2, w21[...], preferred_element_type=jnp.float32) + b21[...], 0.0)
    o2[...] = jnp.dot(h2, w22[...], preferred_element_type=jnp.float32) + b22[...]
    h3 = jnp.maximum(jnp.dot(ef4, w31[...], preferred_element_type=jnp.float32) + b31[...], 0.0)
    o3[...] = jnp.dot(h3, w32[...], preferred_element_type=jnp.float32) + b32[...]


def _edge_mlps(ef, w11, b11, w12, b12, w21, b21, w22, b22, w31, b31, w32, b32,
               x, w_in):
    be = 4000
    ef2 = ef.reshape(_E // 2, 2 * _DE)
    ef4 = ef.reshape(_E // 4, 4 * _DE)
    tile2 = lambda b: jnp.tile(b, 2).reshape(1, -1)
    tile4 = lambda b: jnp.tile(b, 4).reshape(1, -1)
    full = lambda shp: pl.BlockSpec(shp, lambda i: (0, 0))
    return pl.pallas_call(
        _edge_mlps_body,
        grid=(_E // be,),
        in_specs=[
            pl.BlockSpec((be // 2, 2 * _DE), lambda i: (i, 0)),
            pl.BlockSpec((be // 4, 4 * _DE), lambda i: (i, 0)),
            full((2 * _DE, 128)), full((1, 128)), full((128, 128)), full((1, 128)),
            full((2 * _DE, 128)), full((1, 128)), full((128, 128)), full((1, 128)),
            full((4 * _DE, 128)), full((1, 128)), full((128, 128)), full((1, 128)),
            full((_N, _DN)), full((_DN, _H)),
        ],
        out_specs=[
            pl.BlockSpec((be * _H // 128, 128), lambda i: (i, 0)),
            pl.BlockSpec((be * _H // 128, 128), lambda i: (i, 0)),
            pl.BlockSpec((be * _OUT // 128, 128), lambda i: (i, 0)),
            full((_N, _H)),
        ],
        out_shape=[
            jax.ShapeDtypeStruct((_E * _H // 128, 128), jnp.float32),
            jax.ShapeDtypeStruct((_E * _H // 128, 128), jnp.float32),
            jax.ShapeDtypeStruct((_E * _OUT // 128, 128), jnp.float32),
            jax.ShapeDtypeStruct((_N, _H), jnp.float32),
        ],
    )(ef2, ef4,
      _blockdiag(w11, 2), tile2(b11), _blockdiag(w12, 2), tile2(b12),
      _blockdiag(w21, 2), tile2(b21), _blockdiag(w22, 2), tile2(b22),
      _blockdiag(w31, 4), tile4(b31), _blockdiag(w32, 4), tile4(b32),
      x, w_in)


def _mm_body(x_ref, w_ref, o_ref):
    o_ref[...] = jnp.dot(x_ref[...], w_ref[...], preferred_element_type=jnp.float32)


def _matmul(x, w):
    n, d = x.shape
    h = w.shape[1]
    return pl.pallas_call(
        _mm_body,
        out_shape=jax.ShapeDtypeStruct((n, h), jnp.float32),
    )(x, w)


def _relu_mm_body(ya_ref, yb_ref, b_ref, w_ref, o_ref):
    x = jnp.maximum(ya_ref[...] + yb_ref[...] + b_ref[...], 0.0)
    o_ref[...] = jnp.dot(x, w_ref[...], preferred_element_type=jnp.float32)


def _relu_mm(ya, yb, b, w):
    n, h = ya.shape
    h2 = w.shape[1]
    return pl.pallas_call(
        _relu_mm_body,
        out_shape=jax.ShapeDtypeStruct((n, h2), jnp.float32),
    )(ya, yb, b.reshape(1, -1), w)


def _set2set_body(ya_ref, yb_ref, bo_ref, batch_ref, wi_ref, wh_ref, bb_ref, o_ref):
    x = ya_ref[...] + yb_ref[...] + bo_ref[...]
    wi = wi_ref[...]
    wh = wh_ref[...]
    bb = bb_ref[...]
    batch = batch_ref[...]
    seg = lax.broadcasted_iota(jnp.int32, (_B, _N), 0)
    mask = jnp.broadcast_to(batch, (_B, _N)) == seg

    q_star = jnp.zeros((_B, 2 * _OUT), jnp.float32)
    h = jnp.zeros((_B, _OUT), jnp.float32)
    c = jnp.zeros((_B, _OUT), jnp.float32)
    for t in range(_T):
        gates = (jnp.dot(q_star, wi, preferred_element_type=jnp.float32)
                 + jnp.dot(h, wh, preferred_element_type=jnp.float32) + bb)
        ig = jax.nn.sigmoid(gates[:, 0 * _OUT:1 * _OUT])
        fg = jax.nn.sigmoid(gates[:, 1 * _OUT:2 * _OUT])
        gg = jnp.tanh(gates[:, 2 * _OUT:3 * _OUT])
        og = jax.nn.sigmoid(gates[:, 3 * _OUT:4 * _OUT])
        c = fg * c + ig * gg
        h = og * jnp.tanh(c)
        if t == _T - 1:
            break
        s = lax.dot_general(h, x, (((1,), (1,)), ((), ())),
                            preferred_element_type=jnp.float32)
        sm = jnp.where(mask, s, -1e30)
        m = jnp.max(sm, axis=1, keepdims=True)
        ex = jnp.exp(jnp.where(mask, s - m, -1e30))
        den = jnp.sum(ex, axis=1, keepdims=True)
        a = ex / (den + 1e-16)
        r = jnp.dot(a, x, preferred_element_type=jnp.float32)
        q_star = jnp.concatenate([h, r], axis=1)
    o_ref[...] = h


def _set2set(ya, yb, b_out, batch, wi, wh, bsum):
    return pl.pallas_call(
        _set2set_body,
        out_shape=jax.ShapeDtypeStruct((_B, _OUT), jnp.float32),
    )(ya, yb, b_out.reshape(1, -1), batch.reshape(1, -1), wi, wh,
      bsum.reshape(1, -1))


@functools.lru_cache(maxsize=None)
def _make_egc_sc(h):
    rows_per_sub = 632
    base_chunks = _NCHUNKS // _NW
    rem = _NCHUNKS % _NW
    nlmax = base_chunks + 1
    mesh = plsc.VectorSubcoreMesh(core_axis_name="c", subcore_axis_name="s",
                                  num_cores=_NC, num_subcores=_NS)

    @functools.partial(
        pl.kernel,
        out_type=jax.ShapeDtypeStruct((_NC, _N, h), jnp.float32),
        mesh=mesh,
        scratch_types=[
            pltpu.VMEM((nlmax, _C), jnp.int32),
            pltpu.VMEM((nlmax, _C), jnp.int32),
            pltpu.VMEM((h, 128), jnp.float32),
            pltpu.VMEM((h, 128), jnp.float32),
            pltpu.VMEM((_C, h), jnp.float32),
            pltpu.VMEM((_C, h), jnp.float32),
            pltpu.VMEM_SHARED((_N, h), jnp.float32),
            pltpu.SemaphoreType.DMA,
            pltpu.SemaphoreType.DMA,
            pltpu.SemaphoreType.DMA,
            pltpu.SemaphoreType.DMA,
        ],
        compiler_params=pltpu.CompilerParams(use_tc_tiling_on_sc=False),
    )
    def egc(xw_hbm, ef_hbm, esrc_hbm, etgt_hbm, zeros_hbm, out_hbm,
            esrc_l, etgt_l, ef0, ef1, rows0, rows1, acc_sh,
            sem_e0, sem_e1, sem_g0, sem_g1):
        ef_b = (ef0, ef1)
        rows_b = (rows0, rows1)
        sem_e = (sem_e0, sem_e1)
        sem_g = (sem_g0, sem_g1)
        cid = lax.axis_index("c")
        sid = lax.axis_index("s")
        wid = sid * _NC + cid
        row0 = pl.multiple_of(jnp.minimum(sid * rows_per_sub, _N - rows_per_sub), 8)
        pltpu.sync_copy(zeros_hbm.at[pl.ds(row0, rows_per_sub)],
                        acc_sh.at[pl.ds(row0, rows_per_sub)])

        nloc = base_chunks + jnp.where(wid < rem, 1, 0)
        c0 = wid * base_chunks + jnp.minimum(wid, rem)
        c0r = jnp.minimum(c0, _NCHUNKS - nlmax)
        off = c0 - c0r
        pltpu.sync_copy(esrc_hbm.at[pl.ds(c0r, nlmax)], esrc_l)
        pltpu.sync_copy(etgt_hbm.at[pl.ds(c0r, nlmax)], etgt_l)
        plsc.subcore_barrier()

        epr = 128 // h

        def fire(j, b):
            @pl.when(j < nloc)
            def _():
                pltpu.async_copy(ef_hbm.at[pl.ds((c0 + j) * h, h)], ef_b[b],
                                 sem_e[b])
                pltpu.async_copy(xw_hbm.at[esrc_l.at[j + off]], rows_b[b],
                                 sem_g[b])

        def process(j, b):
            @pl.when(j < nloc)
            def _():
                pltpu.make_async_copy(ef_hbm.at[pl.ds(0, h)], ef_b[b],
                                      sem_e[b]).wait()
                pltpu.make_async_copy(xw_hbm.at[esrc_l.at[j + off]],
                                      rows_b[b], sem_g[b]).wait()

                @plsc.parallel_loop(0, h, 1, unroll=2)
                def _mul(i):
                    for p in range(epr):
                        for h0 in range(h // 16):
                            sl = pl.ds(h0 * 16, 16)
                            esl = pl.ds(p * h + h0 * 16, 16)
                            rows_b[b][i * epr + p, sl] = (
                                rows_b[b][i * epr + p, sl] * ef_b[b][i, esl])

                pltpu.sync_copy(rows_b[b], acc_sh.at[etgt_l.at[j + off]],
                                add=True)

        fire(0, 0)

        def outer(i, carry):
            j = 2 * i
            fire(j + 1, 1)
            process(j, 0)
            fire(j + 2, 0)
            process(j + 1, 1)
            return carry

        lax.fori_loop(0, (nlmax + 1) // 2, outer, 0)
        plsc.subcore_barrier()
        pltpu.sync_copy(acc_sh.at[pl.ds(row0, rows_per_sub)],
                        out_hbm.at[cid, pl.ds(row0, rows_per_sub)])

    return egc


def kernel(node_features, edge_features, Esrc, Etgt, batch,
           W_in, b_in, W_mid, b_mid, W_out, b_out,
           ee1_w1, ee1_b1, ee1_w2, ee1_b2,
           ee2_w1, ee2_b1, ee2_w2, ee2_b2,
           ee3_w1, ee3_b1, ee3_w2, ee3_b2,
           Wi, Wh, bi, bh):
    efin, efmid, efout, xw0 = _edge_mlps(
        edge_features,
        ee1_w1, ee1_b1, ee1_w2, ee1_b2,
        ee2_w1, ee2_b1, ee2_w2, ee2_b2,
        ee3_w1, ee3_b1, ee3_w2, ee3_b2,
        node_features, W_in)

    zeros_h = jnp.zeros((_N, _H), jnp.float32)
    zeros_o = jnp.zeros((_N, _OUT), jnp.float32)
    esrc2 = Esrc.reshape(_NCHUNKS, _C)
    etgt2 = Etgt.reshape(_NCHUNKS, _C)

    y1 = _make_egc_sc(_H)(xw0, efin, esrc2, etgt2, zeros_h)
    xw = _relu_mm(y1[0], y1[1], b_in, W_mid)
    y2 = _make_egc_sc(_H)(xw, efmid, esrc2, etgt2, zeros_h)
    xw = _relu_mm(y2[0], y2[1], b_mid, W_out)
    y3 = _make_egc_sc(_OUT)(xw, efout, esrc2, etgt2, zeros_o)
    return _set2set(y3[0], y3[1], b_out, batch, Wi, Wh, bi + bh)


# speedup vs baseline: 1.4106x; 1.0257x over previous
import functools

import jax
import jax.numpy as jnp
from jax import lax
from jax.experimental import pallas as pl
from jax.experimental.pallas import tpu as pltpu
from jax.experimental.pallas import tpu_sc as plsc

_N = 10000
_E = 320000
_B = 64
_DN = 128
_DE = 16
_H = 64
_OUT = 32
_T = 12

_NC = 2
_NS = 16
_NW = _NC * _NS
_C = 128
_NCHUNKS = _E // _C


def _blockdiag(w, k):
    d, h = w.shape
    out = jnp.zeros((k * d, k * h), jnp.float32)
    for i in range(k):
        out = out.at[i * d:(i + 1) * d, i * h:(i + 1) * h].set(w)
    return out


def _edge_mlps_body(ef2_ref, ef4_ref, w11, b11, w12, b12, w21, b21, w22, b22,
                    w31, b31, w32, b32, x_ref, win_ref, o1, o2, o3, oxw):
    @pl.when(pl.program_id(0) == 0)
    def _():
        oxw[...] = jnp.dot(x_ref[...], win_ref[...],
                           preferred_element_type=jnp.float32)

    ef2 = ef2_ref[...]
    ef4 = ef4_ref[...]
    h1 = jnp.maximum(jnp.dot(ef2, w11[...], preferred_element_type=jnp.float32) + b11[...], 0.0)
    o1[...] = jnp.dot(h1, w12[...], preferred_element_type=jnp.float32) + b12[...]
    h2 = jnp.maximum(jnp.dot(ef2, w21[...], preferred_element_type=jnp.float32) + b21[...], 0.0)
    o2[...] = jnp.dot(h2, w22[...], preferred_element_type=jnp.float32) + b22[...]
    h3 = jnp.maximum(jnp.dot(ef4, w31[...], preferred_element_type=jnp.float32) + b31[...], 0.0)
    o3[...] = jnp.dot(h3, w32[...], preferred_element_type=jnp.float32) + b32[...]


def _edge_mlps(ef, w11, b11, w12, b12, w21, b21, w22, b22, w31, b31, w32, b32,
               x, w_in):
    be = 4000
    ef2 = ef.reshape(_E // 2, 2 * _DE)
    ef4 = ef.reshape(_E // 4, 4 * _DE)
    tile2 = lambda b: jnp.tile(b, 2).reshape(1, -1)
    tile4 = lambda b: jnp.tile(b, 4).reshape(1, -1)
    full = lambda shp: pl.BlockSpec(shp, lambda i: (0, 0))
    return pl.pallas_call(
        _edge_mlps_body,
        grid=(_E // be,),
        in_specs=[
            pl.BlockSpec((be // 2, 2 * _DE), lambda i: (i, 0)),
            pl.BlockSpec((be // 4, 4 * _DE), lambda i: (i, 0)),
            full((2 * _DE, 128)), full((1, 128)), full((128, 128)), full((1, 128)),
            full((2 * _DE, 128)), full((1, 128)), full((128, 128)), full((1, 128)),
            full((4 * _DE, 128)), full((1, 128)), full((128, 128)), full((1, 128)),
            full((_N, _DN)), full((_DN, _H)),
        ],
        out_specs=[
            pl.BlockSpec((be * _H // 128, 128), lambda i: (i, 0)),
            pl.BlockSpec((be * _H // 128, 128), lambda i: (i, 0)),
            pl.BlockSpec((be * _OUT // 128, 128), lambda i: (i, 0)),
            full((_N, _H)),
        ],
        out_shape=[
            jax.ShapeDtypeStruct((_E * _H // 128, 128), jnp.float32),
            jax.ShapeDtypeStruct((_E * _H // 128, 128), jnp.float32),
            jax.ShapeDtypeStruct((_E * _OUT // 128, 128), jnp.float32),
            jax.ShapeDtypeStruct((_N, _H), jnp.float32),
        ],
    )(ef2, ef4,
      _blockdiag(w11, 2), tile2(b11), _blockdiag(w12, 2), tile2(b12),
      _blockdiag(w21, 2), tile2(b21), _blockdiag(w22, 2), tile2(b22),
      _blockdiag(w31, 4), tile4(b31), _blockdiag(w32, 4), tile4(b32),
      x, w_in)


def _mm_body(x_ref, w_ref, o_ref):
    o_ref[...] = jnp.dot(x_ref[...], w_ref[...], preferred_element_type=jnp.float32)


def _matmul(x, w):
    n, d = x.shape
    h = w.shape[1]
    return pl.pallas_call(
        _mm_body,
        out_shape=jax.ShapeDtypeStruct((n, h), jnp.float32),
    )(x, w)


def _relu_mm_body(y_ref, b_ref, w_ref, o_ref):
    x = jnp.maximum(y_ref[0] + y_ref[1] + b_ref[...], 0.0)
    o_ref[...] = jnp.dot(x, w_ref[...], preferred_element_type=jnp.float32)


def _relu_mm(y, b, w):
    n, h = y.shape[1], y.shape[2]
    h2 = w.shape[1]
    return pl.pallas_call(
        _relu_mm_body,
        out_shape=jax.ShapeDtypeStruct((n, h2), jnp.float32),
    )(y, b.reshape(1, -1), w)


def _set2set_body(y_ref, bo_ref, batch_ref, wi_ref, wh_ref, bb_ref, o_ref):
    x = y_ref[0] + y_ref[1] + bo_ref[...]
    wi = wi_ref[...]
    wh = wh_ref[...]
    bb = bb_ref[...]
    batch = batch_ref[...]
    seg = lax.broadcasted_iota(jnp.int32, (_B, _N), 0)
    mask = jnp.broadcast_to(batch, (_B, _N)) == seg

    q_star = jnp.zeros((_B, 2 * _OUT), jnp.float32)
    h = jnp.zeros((_B, _OUT), jnp.float32)
    c = jnp.zeros((_B, _OUT), jnp.float32)
    for t in range(_T):
        gates = (jnp.dot(q_star, wi, preferred_element_type=jnp.float32)
                 + jnp.dot(h, wh, preferred_element_type=jnp.float32) + bb)
        ig = jax.nn.sigmoid(gates[:, 0 * _OUT:1 * _OUT])
        fg = jax.nn.sigmoid(gates[:, 1 * _OUT:2 * _OUT])
        gg = jnp.tanh(gates[:, 2 * _OUT:3 * _OUT])
        og = jax.nn.sigmoid(gates[:, 3 * _OUT:4 * _OUT])
        c = fg * c + ig * gg
        h = og * jnp.tanh(c)
        if t == _T - 1:
            break
        s = lax.dot_general(h, x, (((1,), (1,)), ((), ())),
                            preferred_element_type=jnp.float32)
        sm = jnp.where(mask, s, -1e30)
        m = jnp.max(sm, axis=1, keepdims=True)
        ex = jnp.exp(jnp.where(mask, s - m, -1e30))
        den = jnp.sum(ex, axis=1, keepdims=True)
        a = ex / (den + 1e-16)
        r = jnp.dot(a, x, preferred_element_type=jnp.float32)
        q_star = jnp.concatenate([h, r], axis=1)
    o_ref[...] = h


def _set2set(y, b_out, batch, wi, wh, bsum):
    return pl.pallas_call(
        _set2set_body,
        out_shape=jax.ShapeDtypeStruct((_B, _OUT), jnp.float32),
    )(y, b_out.reshape(1, -1), batch.reshape(1, -1), wi, wh,
      bsum.reshape(1, -1))


@functools.lru_cache(maxsize=None)
def _make_egc_sc(h):
    rows_per_sub = 632
    base_chunks = _NCHUNKS // _NW
    rem = _NCHUNKS % _NW
    nlmax = base_chunks + 1
    mesh = plsc.VectorSubcoreMesh(core_axis_name="c", subcore_axis_name="s",
                                  num_cores=_NC, num_subcores=_NS)

    @functools.partial(
        pl.kernel,
        out_type=jax.ShapeDtypeStruct((_NC, _N, h), jnp.float32),
        mesh=mesh,
        scratch_types=[
            pltpu.VMEM((nlmax, _C), jnp.int32),
            pltpu.VMEM((nlmax, _C), jnp.int32),
            pltpu.VMEM((h, 128), jnp.float32),
            pltpu.VMEM((h, 128), jnp.float32),
            pltpu.VMEM((_C, h), jnp.float32),
            pltpu.VMEM((_C, h), jnp.float32),
            pltpu.VMEM_SHARED((_N, h), jnp.float32),
            pltpu.SemaphoreType.DMA,
            pltpu.SemaphoreType.DMA,
            pltpu.SemaphoreType.DMA,
            pltpu.SemaphoreType.DMA,
        ],
        compiler_params=pltpu.CompilerParams(use_tc_tiling_on_sc=False),
    )
    def egc(xw_hbm, ef_hbm, esrc_hbm, etgt_hbm, zeros_hbm, out_hbm,
            esrc_l, etgt_l, ef0, ef1, rows0, rows1, acc_sh,
            sem_e0, sem_e1, sem_g0, sem_g1):
        ef_b = (ef0, ef1)
        rows_b = (rows0, rows1)
        sem_e = (sem_e0, sem_e1)
        sem_g = (sem_g0, sem_g1)
        cid = lax.axis_index("c")
        sid = lax.axis_index("s")
        wid = sid * _NC + cid
        row0 = pl.multiple_of(jnp.minimum(sid * rows_per_sub, _N - rows_per_sub), 8)
        pltpu.sync_copy(zeros_hbm.at[pl.ds(row0, rows_per_sub)],
                        acc_sh.at[pl.ds(row0, rows_per_sub)])

        nloc = base_chunks + jnp.where(wid < rem, 1, 0)
        c0 = wid * base_chunks + jnp.minimum(wid, rem)
        c0r = jnp.minimum(c0, _NCHUNKS - nlmax)
        off = c0 - c0r
        pltpu.sync_copy(esrc_hbm.at[pl.ds(c0r, nlmax)], esrc_l)
        pltpu.sync_copy(etgt_hbm.at[pl.ds(c0r, nlmax)], etgt_l)
        plsc.subcore_barrier()

        epr = 128 // h

        def fire(j, b):
            @pl.when(j < nloc)
            def _():
                pltpu.async_copy(ef_hbm.at[pl.ds((c0 + j) * h, h)], ef_b[b],
                                 sem_e[b])
                pltpu.async_copy(xw_hbm.at[esrc_l.at[j + off]], rows_b[b],
                                 sem_g[b])

        def process(j, b):
            @pl.when(j < nloc)
            def _():
                pltpu.make_async_copy(ef_hbm.at[pl.ds(0, h)], ef_b[b],
                                      sem_e[b]).wait()
                pltpu.make_async_copy(xw_hbm.at[esrc_l.at[j + off]],
                                      rows_b[b], sem_g[b]).wait()

                @plsc.parallel_loop(0, h, 1, unroll=2)
                def _mul(i):
                    for p in range(epr):
                        for h0 in range(h // 16):
                            sl = pl.ds(h0 * 16, 16)
                            esl = pl.ds(p * h + h0 * 16, 16)
                            rows_b[b][i * epr + p, sl] = (
                                rows_b[b][i * epr + p, sl] * ef_b[b][i, esl])

                pltpu.sync_copy(rows_b[b], acc_sh.at[etgt_l.at[j + off]],
                                add=True)

        fire(0, 0)

        def outer(i, carry):
            j = 2 * i
            fire(j + 1, 1)
            process(j, 0)
            fire(j + 2, 0)
            process(j + 1, 1)
            return carry

        lax.fori_loop(0, (nlmax + 1) // 2, outer, 0)
        plsc.subcore_barrier()
        pltpu.sync_copy(acc_sh.at[pl.ds(row0, rows_per_sub)],
                        out_hbm.at[cid, pl.ds(row0, rows_per_sub)])

    return egc


def kernel(node_features, edge_features, Esrc, Etgt, batch,
           W_in, b_in, W_mid, b_mid, W_out, b_out,
           ee1_w1, ee1_b1, ee1_w2, ee1_b2,
           ee2_w1, ee2_b1, ee2_w2, ee2_b2,
           ee3_w1, ee3_b1, ee3_w2, ee3_b2,
           Wi, Wh, bi, bh):
    efin, efmid, efout, xw0 = _edge_mlps(
        edge_features,
        ee1_w1, ee1_b1, ee1_w2, ee1_b2,
        ee2_w1, ee2_b1, ee2_w2, ee2_b2,
        ee3_w1, ee3_b1, ee3_w2, ee3_b2,
        node_features, W_in)

    zeros_h = jnp.zeros((_N, _H), jnp.float32)
    zeros_o = jnp.zeros((_N, _OUT), jnp.float32)
    esrc2 = Esrc.reshape(_NCHUNKS, _C)
    etgt2 = Etgt.reshape(_NCHUNKS, _C)

    y1 = _make_egc_sc(_H)(xw0, efin, esrc2, etgt2, zeros_h)
    xw = _relu_mm(y1, b_in, W_mid)
    y2 = _make_egc_sc(_H)(xw, efmid, esrc2, etgt2, zeros_h)
    xw = _relu_mm(y2, b_mid, W_out)
    y3 = _make_egc_sc(_OUT)(xw, efout, esrc2, etgt2, zeros_o)
    return _set2set(y3, b_out, batch, Wi, Wh, bi + bh)
